# baseline (speedup 1.0000x reference)
import functools

import jax
import jax.numpy as jnp
from jax import lax
from jax.experimental import pallas as pl
from jax.experimental.pallas import tpu as pltpu
from jax.experimental.pallas import tpu_sc as plsc

_LP = 8


def kernel(idxs, locs, data):
    B, N, D = locs.shape
    C = data.shape[2]
    RT = B * N

    CHUNK = 128
    assert RT % CHUNK == 0
    NCHUNKS = RT // CHUNK

    info = plsc.get_sparse_core_info()
    NC, NS = info.num_cores, info.num_subcores
    NW = NC * NS
    ITERS = (NCHUNKS + NW - 1) // NW

    idxs_flat = idxs.reshape(RT)
    locs2d = jnp.pad(locs.reshape(RT, D), ((0, 0), (0, _LP - D)))
    data2d = data.reshape(RT, C)

    mesh = plsc.VectorSubcoreMesh(core_axis_name="c", subcore_axis_name="s")

    @functools.partial(
        pl.kernel,
        mesh=mesh,
        out_type=[
            jax.ShapeDtypeStruct((RT, D), jnp.float32),
            jax.ShapeDtypeStruct((RT, C), jnp.float32),
        ],
        scratch_types=[
            pltpu.VMEM((CHUNK,), jnp.int32),
            pltpu.VMEM((CHUNK, _LP), jnp.float32),
            pltpu.VMEM((CHUNK, C), jnp.float32),
            pltpu.SemaphoreType.DMA,
        ],
        compiler_params=pltpu.CompilerParams(use_tc_tiling_on_sc=False),
    )
    def k(idx_hbm, locs_hbm, data_hbm, outl_hbm, outd_hbm, idxbuf, lbuf, dbuf, sem):
        wid = lax.axis_index("s") * NC + lax.axis_index("c")

        def body(t, carry):
            c = wid + NW * t

            @pl.when(c < NCHUNKS)
            def _():
                r = c * CHUNK
                pltpu.sync_copy(idx_hbm.at[pl.ds(r, CHUNK)], idxbuf)
                for i in range(CHUNK // 16):
                    pos = r + i * 16 + lax.iota(jnp.int32, 16)
                    base = lax.div(pos, N) * N
                    idxbuf[pl.ds(i * 16, 16)] = idxbuf[pl.ds(i * 16, 16)] + base
                gd = pltpu.async_copy(data_hbm.at[idxbuf], dbuf, sem)
                gl = pltpu.async_copy(locs_hbm.at[idxbuf], lbuf, sem)
                gd.wait()
                gl.wait()
                pltpu.sync_copy(dbuf, outd_hbm.at[pl.ds(r, CHUNK)])
                pltpu.sync_copy(lbuf.at[:, pl.ds(0, D)], outl_hbm.at[pl.ds(r, CHUNK)])

            return carry

        lax.fori_loop(0, ITERS, body, 0)

    outl, outd = k(idxs_flat, locs2d, data2d)
    return (outl.reshape(B, N, D), outd.reshape(B, N, C))
